# baseline (speedup 1.0000x reference)
import functools

import jax
import jax.numpy as jnp
from jax import lax
from jax.experimental import pallas as pl
from jax.experimental.pallas import tpu as pltpu
from jax.experimental.pallas import tpu_sc as plsc

_NC = 2
_NS = 16
_NW = _NC * _NS
_CHUNK = 128


def _deg_sc(dst2d, ew2d, n_pad):
    rows_total = dst2d.shape[0]
    cpt = rows_total // _NW
    rpt = n_pad // _NS
    nzc = rpt // _CHUNK

    mesh = plsc.VectorSubcoreMesh(core_axis_name="c", subcore_axis_name="s")

    @functools.partial(
        pl.kernel,
        out_type=jax.ShapeDtypeStruct((_NC * n_pad // 8, 128), jnp.float32),
        mesh=mesh,
        scratch_types=[
            pltpu.VMEM((cpt, _CHUNK), jnp.int32),
            pltpu.VMEM((cpt, _CHUNK), jnp.float32),
            pltpu.VMEM((_CHUNK,), jnp.float32),
            pltpu.VMEM((rpt,), jnp.float32),
            pltpu.VMEM((rpt // 8, 128), jnp.float32),
            pltpu.VMEM_SHARED((n_pad,), jnp.float32),
        ] + [pltpu.SemaphoreType.DMA] * 8,
        compiler_params=pltpu.CompilerParams(use_tc_tiling_on_sc=False),
    )
    def k(dst_hbm, ew_hbm, out_hbm, dst_v, ew_v, zbuf, degbuf, repbuf,
          acc_sh, *sems):
        c = lax.axis_index("c")
        s = lax.axis_index("s")
        wid = c * _NS + s
        base = wid * cpt

        def zfill(j, _):
            zbuf[pl.ds(j * 16, 16)] = jnp.zeros((16,), jnp.float32)
            return 0
        lax.fori_loop(0, _CHUNK // 16, zfill, 0)

        def zcopy(j, _):
            pltpu.sync_copy(zbuf, acc_sh.at[pl.ds(s * rpt + j * _CHUNK, _CHUNK)])
            return 0
        lax.fori_loop(0, nzc, zcopy, 0)

        pltpu.sync_copy(dst_hbm.at[pl.ds(base, cpt)], dst_v)
        pltpu.sync_copy(ew_hbm.at[pl.ds(base, cpt)], ew_v)

        plsc.subcore_barrier()

        def oct8(q, _):
            for k8 in range(8):
                i = 8 * q + k8

                @pl.when(q >= 1)
                def _():
                    pltpu.make_async_copy(
                        ew_v.at[i - 8], acc_sh.at[dst_v.at[i - 8]],
                        sems[k8]).wait()

                pltpu.async_copy(ew_v.at[i], acc_sh.at[dst_v.at[i]], sems[k8],
                                 add=True)
            return 0
        lax.fori_loop(0, cpt // 8, oct8, 0)
        for k8 in range(8):
            i = cpt - 8 + k8
            pltpu.make_async_copy(ew_v.at[i], acc_sh.at[dst_v.at[i]],
                                  sems[k8]).wait()

        plsc.subcore_barrier()

        pltpu.sync_copy(acc_sh.at[pl.ds(s * rpt, rpt)], degbuf)

        def rep(g, _):
            d16 = degbuf[pl.ds(g * 16, 16)]
            for l in range(16):
                repbuf[2 * g + l // 8, pl.ds(16 * (l % 8), 16)] = (
                    jnp.full((16,), d16[l], jnp.float32))
            return 0
        lax.fori_loop(0, rpt // 16, rep, 0)

        pltpu.sync_copy(repbuf,
                        out_hbm.at[pl.ds((c * n_pad + s * rpt) // 8, rpt // 8)])

    return k(dst2d, ew2d)


def _agg_sc(src2d, dst2d, ew2d, table, n_pad):
    rows_total = src2d.shape[0]
    cpt = rows_total // _NW
    rpt = n_pad // _NS
    nzc = rpt // _CHUNK

    mesh = plsc.VectorSubcoreMesh(core_axis_name="c", subcore_axis_name="s")

    @functools.partial(
        pl.kernel,
        out_type=jax.ShapeDtypeStruct((_NC * n_pad // 8, 128), jnp.float32),
        mesh=mesh,
        scratch_types=[
            pltpu.VMEM((cpt, _CHUNK), jnp.int32),
            pltpu.VMEM((cpt, _CHUNK), jnp.int32),
            pltpu.VMEM((cpt, _CHUNK), jnp.float32),
            pltpu.VMEM((4, _CHUNK, 16), jnp.float32),
            pltpu.VMEM((rpt // 8, 128), jnp.float32),
            pltpu.VMEM((rpt, 16), jnp.float32),
            pltpu.VMEM_SHARED((n_pad, 16), jnp.float32),
            pltpu.VMEM_SHARED((n_pad, 16), jnp.float32),
        ] + [pltpu.SemaphoreType.DMA] * 8,
        compiler_params=pltpu.CompilerParams(use_tc_tiling_on_sc=False),
    )
    def k(src_hbm, dst_hbm, ew_hbm, tab_hbm, out_hbm,
          src_v, dst_v, ew_v, rows_v, b128, b16, acc_sh, tab_sh, *sems):
        gsem = sems[0:4]
        ssem = sems[4:8]
        c = lax.axis_index("c")
        s = lax.axis_index("s")
        wid = c * _NS + s
        base = wid * cpt

        def wide_to_rows(g, _):
            for l in range(16):
                b16[g * 16 + l, :] = b128[2 * g + l // 8,
                                          pl.ds(16 * (l % 8), 16)]
            return 0

        def rows_to_wide(g, _):
            for l in range(16):
                b128[2 * g + l // 8, pl.ds(16 * (l % 8), 16)] = (
                    b16[g * 16 + l, :])
            return 0

        def zrow(e, _):
            rows_v[0, e, :] = jnp.zeros((16,), jnp.float32)
            return 0
        lax.fori_loop(0, _CHUNK, zrow, 0)

        def zcopy(j, _):
            pltpu.sync_copy(rows_v.at[0],
                            acc_sh.at[pl.ds(s * rpt + j * _CHUNK, _CHUNK)])
            return 0
        lax.fori_loop(0, nzc, zcopy, 0)

        pltpu.sync_copy(tab_hbm.at[pl.ds(s * rpt // 8, rpt // 8)], b128)
        lax.fori_loop(0, rpt // 16, wide_to_rows, 0)
        pltpu.sync_copy(b16, tab_sh.at[pl.ds(s * rpt, rpt)])

        pltpu.sync_copy(src_hbm.at[pl.ds(base, cpt)], src_v)
        pltpu.sync_copy(dst_hbm.at[pl.ds(base, cpt)], dst_v)
        pltpu.sync_copy(ew_hbm.at[pl.ds(base, cpt)], ew_v)

        plsc.subcore_barrier()

        for b in range(3):
            pltpu.async_copy(tab_sh.at[src_v.at[b]], rows_v.at[b], gsem[b])

        def quad(q, _):
            for b in range(4):
                i = 4 * q + b
                j = i + 3
                bp = (b + 3) % 4

                @pl.when(j < cpt)
                def _():
                    if b == 0:
                        @pl.when(q >= 1)
                        def _():
                            pltpu.make_async_copy(
                                rows_v.at[bp], acc_sh.at[dst_v.at[i - 1]],
                                ssem[bp]).wait()
                    else:
                        pltpu.make_async_copy(
                            rows_v.at[bp], acc_sh.at[dst_v.at[i - 1]],
                            ssem[bp]).wait()
                    pltpu.async_copy(tab_sh.at[src_v.at[j]], rows_v.at[bp],
                                     gsem[bp])

                pltpu.make_async_copy(tab_sh.at[src_v.at[i]], rows_v.at[b],
                                      gsem[b]).wait()

                def mulgrp(g, _):
                    ew16 = ew_v[i, pl.ds(g * 16, 16)]
                    for l in range(16):
                        e0 = g * 16 + l
                        rows_v[b, e0, :] = rows_v[b, e0, :] * ew16[l]
                    return 0
                lax.fori_loop(0, _CHUNK // 16, mulgrp, 0)

                pltpu.async_copy(rows_v.at[b], acc_sh.at[dst_v.at[i]],
                                 ssem[b], add=True)
            return 0
        lax.fori_loop(0, cpt // 4, quad, 0)
        for b in range(4):
            i = cpt - 4 + b
            pltpu.make_async_copy(rows_v.at[b], acc_sh.at[dst_v.at[i]],
                                  ssem[b]).wait()

        plsc.subcore_barrier()

        pltpu.sync_copy(acc_sh.at[pl.ds(s * rpt, rpt)], b16)
        lax.fori_loop(0, rpt // 16, rows_to_wide, 0)
        pltpu.sync_copy(b128,
                        out_hbm.at[pl.ds((c * n_pad + s * rpt) // 8, rpt // 8)])

    return k(src2d, dst2d, ew2d, table)


def _tc1(x3, W1, rep, n_pad):
    nr = x3.shape[0]
    nrp = n_pad // 8

    def body(x_ref, w_ref, rep_ref, g_ref, sc_ref, dinv_ref):
        deg = rep_ref[0:nrp] + rep_ref[nrp:2 * nrp] + 1.0
        good = deg > 0
        dinv = jnp.where(good, lax.rsqrt(jnp.where(good, deg, 1.0)), 0.0)
        dinv_ref[...] = dinv
        if nrp > nr:
            g_ref[nr:nrp, :] = jnp.zeros((nrp - nr, 128), jnp.float32)
            sc_ref[nr:nrp, :] = jnp.zeros((nrp - nr, 128), jnp.float32)
        for q in range(8):
            hq = jnp.dot(x_ref[:, q, :], w_ref[...],
                         preferred_element_type=jnp.float32)
            dq = dinv[0:nr, 16 * q:16 * q + 16]
            g_ref[0:nr, 16 * q:16 * q + 16] = dq * hq
            sc_ref[0:nr, 16 * q:16 * q + 16] = (dq * dq) * hq

    return pl.pallas_call(
        body,
        out_shape=[
            jax.ShapeDtypeStruct((nrp, 128), jnp.float32),
            jax.ShapeDtypeStruct((nrp, 128), jnp.float32),
            jax.ShapeDtypeStruct((nrp, 128), jnp.float32),
        ],
    )(x3, W1, rep)


def _tc2(part1, selfc1, dinv, b1exp, W2big, n_pad):
    nrp = n_pad // 8

    def body(p_ref, sc1_ref, dinv_ref, b1_ref, w2_ref, g2_ref, sc2_ref):
        dinv = dinv_ref[...]
        a1 = (dinv * (p_ref[0:nrp] + p_ref[nrp:2 * nrp])
              + sc1_ref[...] + b1_ref[...])
        z = jnp.maximum(a1, 0.0)
        h2 = jnp.dot(z, w2_ref[...], preferred_element_type=jnp.float32)
        g2_ref[...] = dinv * h2
        sc2_ref[...] = (dinv * dinv) * h2

    return pl.pallas_call(
        body,
        out_shape=[
            jax.ShapeDtypeStruct((nrp, 128), jnp.float32),
            jax.ShapeDtypeStruct((nrp, 128), jnp.float32),
        ],
    )(part1, selfc1, dinv, b1exp, W2big)


def _tc3(part2, selfc2, dinv, b2exp, n, n_pad, n_classes):
    nrp = n_pad // 8

    nr = n // 8

    def body(q_ref, sc2_ref, dinv_ref, b2_ref, out_ref):
        t = (dinv_ref[...] * (q_ref[0:nrp] + q_ref[nrp:2 * nrp])
             + sc2_ref[...] + b2_ref[...])
        for q in range(8):
            lg = t[0:nr, 16 * q:16 * q + n_classes]
            m = jnp.max(lg, axis=1, keepdims=True)
            e = jnp.exp(lg - m)
            lse = jnp.log(jnp.sum(e, axis=1, keepdims=True))
            out_ref[:, q, :] = lg - m - lse

    return pl.pallas_call(
        body,
        out_shape=jax.ShapeDtypeStruct((nr, 8, n_classes), jnp.float32),
    )(part2, selfc2, dinv, b2exp)


def kernel(x, edge_index, edge_weight, W1, b1, W2, b2):
    n, d_feat = x.shape
    e = edge_index.shape[1]
    hidden = W1.shape[1]
    n_classes = W2.shape[1]

    tile_n = _NS * _CHUNK
    n_pad = -(-n // tile_n) * tile_n
    tile_e = _NW * _CHUNK * 8
    e_pad = -(-e // tile_e) * tile_e

    src = jnp.concatenate(
        [edge_index[0].astype(jnp.int32), jnp.zeros((e_pad - e,), jnp.int32)])
    dst = jnp.concatenate(
        [edge_index[1].astype(jnp.int32), jnp.zeros((e_pad - e,), jnp.int32)])
    ew = jnp.concatenate(
        [edge_weight.astype(jnp.float32), jnp.zeros((e_pad - e,), jnp.float32)])
    src2d = src.reshape(e_pad // _CHUNK, _CHUNK)
    dst2d = dst.reshape(e_pad // _CHUNK, _CHUNK)
    ew2d = ew.reshape(e_pad // _CHUNK, _CHUNK)

    b1exp = jnp.tile(b1.astype(jnp.float32), 8).reshape(1, 128)
    b2p = jnp.zeros((16,), jnp.float32).at[:n_classes].set(b2)
    b2exp = jnp.tile(b2p, 8).reshape(1, 128)
    W2p = jnp.zeros((hidden, 16), jnp.float32).at[:, :n_classes].set(W2)
    W2big = jnp.kron(jnp.eye(8, dtype=jnp.float32), W2p)

    x3 = x.astype(jnp.float32).reshape(n // 8, 8, d_feat)

    rep = _deg_sc(dst2d, ew2d, n_pad)
    g1, selfc1, dinv = _tc1(x3, W1, rep, n_pad)
    part1 = _agg_sc(src2d, dst2d, ew2d, g1, n_pad)
    g2, selfc2 = _tc2(part1, selfc1, dinv, b1exp, W2big, n_pad)
    part2 = _agg_sc(src2d, dst2d, ew2d, g2, n_pad)
    out3 = _tc3(part2, selfc2, dinv, b2exp, n, n_pad, n_classes)
    return out3.reshape(n, n_classes)


# speedup vs baseline: 1.0333x; 1.0333x over previous
import functools

import jax
import jax.numpy as jnp
from jax import lax
from jax.experimental import pallas as pl
from jax.experimental.pallas import tpu as pltpu
from jax.experimental.pallas import tpu_sc as plsc

_NC = 2
_NS = 16
_NW = _NC * _NS
_CHUNK = 128


def _deg_sc(dst2d, ew2d, n_pad):
    rows_total = dst2d.shape[0]
    cpt = rows_total // _NW
    rpt = n_pad // _NS
    nzc = rpt // _CHUNK

    mesh = plsc.VectorSubcoreMesh(core_axis_name="c", subcore_axis_name="s")

    @functools.partial(
        pl.kernel,
        out_type=jax.ShapeDtypeStruct((_NC * n_pad // 8, 128), jnp.float32),
        mesh=mesh,
        scratch_types=[
            pltpu.VMEM((cpt, _CHUNK), jnp.int32),
            pltpu.VMEM((cpt, _CHUNK), jnp.float32),
            pltpu.VMEM((_CHUNK,), jnp.float32),
            pltpu.VMEM((rpt,), jnp.float32),
            pltpu.VMEM((rpt // 8, 128), jnp.float32),
            pltpu.VMEM_SHARED((n_pad,), jnp.float32),
        ] + [pltpu.SemaphoreType.DMA] * 8,
        compiler_params=pltpu.CompilerParams(use_tc_tiling_on_sc=False),
    )
    def k(dst_hbm, ew_hbm, out_hbm, dst_v, ew_v, zbuf, degbuf, repbuf,
          acc_sh, *sems):
        c = lax.axis_index("c")
        s = lax.axis_index("s")
        wid = c * _NS + s
        base = wid * cpt

        def zfill(j, _):
            zbuf[pl.ds(j * 16, 16)] = jnp.zeros((16,), jnp.float32)
            return 0
        lax.fori_loop(0, _CHUNK // 16, zfill, 0)

        def zcopy(j, _):
            pltpu.sync_copy(zbuf, acc_sh.at[pl.ds(s * rpt + j * _CHUNK, _CHUNK)])
            return 0
        lax.fori_loop(0, nzc, zcopy, 0)

        pltpu.sync_copy(dst_hbm.at[pl.ds(base, cpt)], dst_v)
        pltpu.sync_copy(ew_hbm.at[pl.ds(base, cpt)], ew_v)

        plsc.subcore_barrier()

        def oct8(q, _):
            for k8 in range(8):
                i = 8 * q + k8

                @pl.when(q >= 1)
                def _():
                    pltpu.make_async_copy(
                        ew_v.at[i - 8], acc_sh.at[dst_v.at[i - 8]],
                        sems[k8]).wait()

                pltpu.async_copy(ew_v.at[i], acc_sh.at[dst_v.at[i]], sems[k8],
                                 add=True)
            return 0
        lax.fori_loop(0, cpt // 8, oct8, 0)
        for k8 in range(8):
            i = cpt - 8 + k8
            pltpu.make_async_copy(ew_v.at[i], acc_sh.at[dst_v.at[i]],
                                  sems[k8]).wait()

        plsc.subcore_barrier()

        pltpu.sync_copy(acc_sh.at[pl.ds(s * rpt, rpt)], degbuf)

        def rep(g, _):
            d16 = degbuf[pl.ds(g * 16, 16)]
            for l in range(16):
                repbuf[2 * g + l // 8, pl.ds(16 * (l % 8), 16)] = (
                    jnp.full((16,), d16[l], jnp.float32))
            return 0
        lax.fori_loop(0, rpt // 16, rep, 0)

        pltpu.sync_copy(repbuf,
                        out_hbm.at[pl.ds((c * n_pad + s * rpt) // 8, rpt // 8)])

    return k(dst2d, ew2d)


def _agg_sc(src2d, dst2d, ew2d, table, n_pad):
    rows_total = src2d.shape[0]
    cpt = rows_total // _NW
    rpt = n_pad // _NS
    nzc = rpt // _CHUNK

    mesh = plsc.VectorSubcoreMesh(core_axis_name="c", subcore_axis_name="s")

    @functools.partial(
        pl.kernel,
        out_type=jax.ShapeDtypeStruct((_NC * n_pad // 8, 128), jnp.float32),
        mesh=mesh,
        scratch_types=[
            pltpu.VMEM((cpt, _CHUNK), jnp.int32),
            pltpu.VMEM((cpt, _CHUNK), jnp.int32),
            pltpu.VMEM((cpt, _CHUNK), jnp.float32),
            pltpu.VMEM((4, _CHUNK, 16), jnp.float32),
            pltpu.VMEM((rpt // 8, 128), jnp.float32),
            pltpu.VMEM((rpt, 16), jnp.float32),
            pltpu.VMEM_SHARED((n_pad, 16), jnp.float32),
            pltpu.VMEM_SHARED((n_pad, 16), jnp.float32),
        ] + [pltpu.SemaphoreType.DMA] * 8,
        compiler_params=pltpu.CompilerParams(use_tc_tiling_on_sc=False),
    )
    def k(src_hbm, dst_hbm, ew_hbm, tab_hbm, out_hbm,
          src_v, dst_v, ew_v, rows_v, b128, b16, acc_sh, tab_sh, *sems):
        gsem = sems[0:4]
        ssem = sems[4:8]
        c = lax.axis_index("c")
        s = lax.axis_index("s")
        wid = c * _NS + s
        base = wid * cpt

        def wide_to_rows(g, _):
            for l in range(16):
                b16[g * 16 + l, :] = b128[2 * g + l // 8,
                                          pl.ds(16 * (l % 8), 16)]
            return 0

        def rows_to_wide(g, _):
            for l in range(16):
                b128[2 * g + l // 8, pl.ds(16 * (l % 8), 16)] = (
                    b16[g * 16 + l, :])
            return 0

        def zrow(e, _):
            rows_v[0, e, :] = jnp.zeros((16,), jnp.float32)
            return 0
        lax.fori_loop(0, _CHUNK, zrow, 0)

        def zcopy(j, _):
            pltpu.sync_copy(rows_v.at[0],
                            acc_sh.at[pl.ds(s * rpt + j * _CHUNK, _CHUNK)])
            return 0
        lax.fori_loop(0, nzc, zcopy, 0)

        pltpu.sync_copy(tab_hbm.at[pl.ds(s * rpt // 8, rpt // 8)], b128)
        lax.fori_loop(0, rpt // 16, wide_to_rows, 0)
        pltpu.sync_copy(b16, tab_sh.at[pl.ds(s * rpt, rpt)])

        pltpu.sync_copy(src_hbm.at[pl.ds(base, cpt)], src_v)
        pltpu.sync_copy(dst_hbm.at[pl.ds(base, cpt)], dst_v)
        pltpu.sync_copy(ew_hbm.at[pl.ds(base, cpt)], ew_v)

        plsc.subcore_barrier()

        for b in range(3):
            pltpu.async_copy(tab_sh.at[src_v.at[b]], rows_v.at[b], gsem[b])

        def quad(q, _):
            for b in range(4):
                i = 4 * q + b
                j = i + 3
                bp = (b + 3) % 4

                @pl.when(j < cpt)
                def _():
                    if b == 0:
                        @pl.when(q >= 1)
                        def _():
                            pltpu.make_async_copy(
                                rows_v.at[bp], acc_sh.at[dst_v.at[i - 1]],
                                ssem[bp]).wait()
                    else:
                        pltpu.make_async_copy(
                            rows_v.at[bp], acc_sh.at[dst_v.at[i - 1]],
                            ssem[bp]).wait()
                    pltpu.async_copy(tab_sh.at[src_v.at[j]], rows_v.at[bp],
                                     gsem[bp])

                pltpu.make_async_copy(tab_sh.at[src_v.at[i]], rows_v.at[b],
                                      gsem[b]).wait()

                def mulgrp(g, _):
                    ew16 = ew_v[i, pl.ds(g * 16, 16)]
                    for l in range(16):
                        e0 = g * 16 + l
                        rows_v[b, e0, :] = rows_v[b, e0, :] * ew16[l]
                    return 0
                lax.fori_loop(0, _CHUNK // 16, mulgrp, 0)

                pltpu.async_copy(rows_v.at[b], acc_sh.at[dst_v.at[i]],
                                 ssem[b], add=True)
            return 0
        lax.fori_loop(0, cpt // 4, quad, 0)
        for b in range(4):
            i = cpt - 4 + b
            pltpu.make_async_copy(rows_v.at[b], acc_sh.at[dst_v.at[i]],
                                  ssem[b]).wait()

        plsc.subcore_barrier()

        pltpu.sync_copy(acc_sh.at[pl.ds(s * rpt, rpt)], b16)
        lax.fori_loop(0, rpt // 16, rows_to_wide, 0)
        pltpu.sync_copy(b128,
                        out_hbm.at[pl.ds((c * n_pad + s * rpt) // 8, rpt // 8)])

    return k(src2d, dst2d, ew2d, table)


def _tc1(x3, W1, rep, n_pad):
    nr = x3.shape[0]
    nrp = n_pad // 8

    def body(x_ref, w_ref, rep_ref, g_ref, sc_ref, dinv_ref):
        deg = rep_ref[0:nrp] + rep_ref[nrp:2 * nrp] + 1.0
        good = deg > 0
        dinv = jnp.where(good, lax.rsqrt(jnp.where(good, deg, 1.0)), 0.0)
        dinv_ref[...] = dinv
        if nrp > nr:
            g_ref[nr:nrp, :] = jnp.zeros((nrp - nr, 128), jnp.float32)
            sc_ref[nr:nrp, :] = jnp.zeros((nrp - nr, 128), jnp.float32)
        h = jnp.concatenate(
            [jnp.dot(x_ref[:, q, :], w_ref[...],
                     preferred_element_type=jnp.float32) for q in range(8)],
            axis=1)
        d = dinv[0:nr]
        g_ref[0:nr, :] = d * h
        sc_ref[0:nr, :] = (d * d) * h

    return pl.pallas_call(
        body,
        out_shape=[
            jax.ShapeDtypeStruct((nrp, 128), jnp.float32),
            jax.ShapeDtypeStruct((nrp, 128), jnp.float32),
            jax.ShapeDtypeStruct((nrp, 128), jnp.float32),
        ],
    )(x3, W1, rep)


def _tc2(part1, selfc1, dinv, b1exp, W2big, n_pad):
    nrp = n_pad // 8

    def body(p_ref, sc1_ref, dinv_ref, b1_ref, w2_ref, g2_ref, sc2_ref):
        dinv = dinv_ref[...]
        a1 = (dinv * (p_ref[0:nrp] + p_ref[nrp:2 * nrp])
              + sc1_ref[...] + b1_ref[...])
        z = jnp.maximum(a1, 0.0)
        h2 = jnp.dot(z, w2_ref[...], preferred_element_type=jnp.float32)
        g2_ref[...] = dinv * h2
        sc2_ref[...] = (dinv * dinv) * h2

    return pl.pallas_call(
        body,
        out_shape=[
            jax.ShapeDtypeStruct((nrp, 128), jnp.float32),
            jax.ShapeDtypeStruct((nrp, 128), jnp.float32),
        ],
    )(part1, selfc1, dinv, b1exp, W2big)


def _tc3(part2, selfc2, dinv, b2exp, n_pad, n_classes):
    nrp = n_pad // 8

    def body(q_ref, sc2_ref, dinv_ref, b2_ref, out_ref):
        t = (dinv_ref[...] * (q_ref[0:nrp] + q_ref[nrp:2 * nrp])
             + sc2_ref[...] + b2_ref[...])
        t3 = t.reshape(nrp, 8, 16)
        lane = lax.broadcasted_iota(jnp.int32, (nrp, 8, 16), 2)
        lg = jnp.where(lane < n_classes, t3, jnp.float32(-1e30))
        m = jnp.max(lg, axis=2, keepdims=True)
        e = jnp.exp(lg - m)
        lse = jnp.log(jnp.sum(e, axis=2, keepdims=True))
        out_ref[...] = (lg - m - lse).reshape(nrp, 128)

    return pl.pallas_call(
        body,
        out_shape=jax.ShapeDtypeStruct((nrp, 128), jnp.float32),
    )(part2, selfc2, dinv, b2exp)


def kernel(x, edge_index, edge_weight, W1, b1, W2, b2):
    n, d_feat = x.shape
    e = edge_index.shape[1]
    hidden = W1.shape[1]
    n_classes = W2.shape[1]

    tile_n = _NS * _CHUNK
    n_pad = -(-n // tile_n) * tile_n
    tile_e = _NW * _CHUNK * 8
    e_pad = -(-e // tile_e) * tile_e

    ei32 = edge_index.astype(jnp.int32)
    src = jnp.concatenate(
        [ei32[0], jnp.zeros((e_pad - e,), jnp.int32)])
    dst = jnp.concatenate(
        [ei32[1], jnp.zeros((e_pad - e,), jnp.int32)])
    ew = jnp.concatenate(
        [edge_weight.astype(jnp.float32), jnp.zeros((e_pad - e,), jnp.float32)])
    src2d = src.reshape(e_pad // _CHUNK, _CHUNK)
    dst2d = dst.reshape(e_pad // _CHUNK, _CHUNK)
    ew2d = ew.reshape(e_pad // _CHUNK, _CHUNK)

    b1exp = jnp.tile(b1.astype(jnp.float32), 8).reshape(1, 128)
    b2p = jnp.zeros((16,), jnp.float32).at[:n_classes].set(b2)
    b2exp = jnp.tile(b2p, 8).reshape(1, 128)
    W2p = jnp.zeros((hidden, 16), jnp.float32).at[:, :n_classes].set(W2)
    W2big = jnp.kron(jnp.eye(8, dtype=jnp.float32), W2p)

    x3 = x.astype(jnp.float32).reshape(n // 8, 8, d_feat)

    rep = _deg_sc(dst2d, ew2d, n_pad)
    g1, selfc1, dinv = _tc1(x3, W1, rep, n_pad)
    part1 = _agg_sc(src2d, dst2d, ew2d, g1, n_pad)
    g2, selfc2 = _tc2(part1, selfc1, dinv, b1exp, W2big, n_pad)
    part2 = _agg_sc(src2d, dst2d, ew2d, g2, n_pad)
    out2d = _tc3(part2, selfc2, dinv, b2exp, n_pad, n_classes)
    return out2d.reshape(n_pad, 16)[:n, :n_classes]


# speedup vs baseline: 1.1450x; 1.1080x over previous
import functools

import jax
import jax.numpy as jnp
from jax import lax
from jax.experimental import pallas as pl
from jax.experimental.pallas import tpu as pltpu
from jax.experimental.pallas import tpu_sc as plsc

_NC = 2
_NS = 16
_NW = _NC * _NS
_CHUNK = 128


def _deg_sc(ei3, ew2d, n_pad):
    rows_total = ei3.shape[1]
    cpt = rows_total // _NW
    rpt = n_pad // _NS
    nzc = rpt // _CHUNK

    mesh = plsc.VectorSubcoreMesh(core_axis_name="c", subcore_axis_name="s")

    @functools.partial(
        pl.kernel,
        out_type=jax.ShapeDtypeStruct((_NC * n_pad // 8, 128), jnp.float32),
        mesh=mesh,
        scratch_types=[
            pltpu.VMEM((cpt, _CHUNK), jnp.int32),
            pltpu.VMEM((cpt, _CHUNK), jnp.float32),
            pltpu.VMEM((_CHUNK,), jnp.float32),
            pltpu.VMEM((rpt,), jnp.float32),
            pltpu.VMEM((rpt // 8, 128), jnp.float32),
            pltpu.VMEM_SHARED((n_pad,), jnp.float32),
        ] + [pltpu.SemaphoreType.DMA] * 8,
        compiler_params=pltpu.CompilerParams(use_tc_tiling_on_sc=False),
    )
    def k(ei_hbm, ew_hbm, out_hbm, dst_v, ew_v, zbuf, degbuf, repbuf,
          acc_sh, *sems):
        c = lax.axis_index("c")
        s = lax.axis_index("s")
        wid = c * _NS + s
        base = wid * cpt

        def zfill(j, _):
            zbuf[pl.ds(j * 16, 16)] = jnp.zeros((16,), jnp.float32)
            return 0
        lax.fori_loop(0, _CHUNK // 16, zfill, 0)

        def zcopy(j, _):
            pltpu.sync_copy(zbuf, acc_sh.at[pl.ds(s * rpt + j * _CHUNK, _CHUNK)])
            return 0
        lax.fori_loop(0, nzc, zcopy, 0)

        pltpu.sync_copy(ei_hbm.at[1, pl.ds(base, cpt)], dst_v)
        pltpu.sync_copy(ew_hbm.at[pl.ds(base, cpt)], ew_v)

        plsc.subcore_barrier()

        def oct8(q, _):
            for k8 in range(8):
                i = 8 * q + k8

                @pl.when(q >= 1)
                def _():
                    pltpu.make_async_copy(
                        ew_v.at[i - 8], acc_sh.at[dst_v.at[i - 8]],
                        sems[k8]).wait()

                pltpu.async_copy(ew_v.at[i], acc_sh.at[dst_v.at[i]], sems[k8],
                                 add=True)
            return 0
        lax.fori_loop(0, cpt // 8, oct8, 0)
        for k8 in range(8):
            i = cpt - 8 + k8
            pltpu.make_async_copy(ew_v.at[i], acc_sh.at[dst_v.at[i]],
                                  sems[k8]).wait()

        plsc.subcore_barrier()

        pltpu.sync_copy(acc_sh.at[pl.ds(s * rpt, rpt)], degbuf)

        def rep(g, _):
            d16 = degbuf[pl.ds(g * 16, 16)]
            for l in range(16):
                repbuf[2 * g + l // 8, pl.ds(16 * (l % 8), 16)] = (
                    jnp.full((16,), d16[l], jnp.float32))
            return 0
        lax.fori_loop(0, rpt // 16, rep, 0)

        pltpu.sync_copy(repbuf,
                        out_hbm.at[pl.ds((c * n_pad + s * rpt) // 8, rpt // 8)])

    return k(ei3, ew2d)


def _agg_sc(ei3, ew2d, table, n_pad):
    rows_total = ei3.shape[1]
    cpt = rows_total // _NW
    rpt = n_pad // _NS
    nzc = rpt // _CHUNK

    mesh = plsc.VectorSubcoreMesh(core_axis_name="c", subcore_axis_name="s")

    @functools.partial(
        pl.kernel,
        out_type=jax.ShapeDtypeStruct((_NC * n_pad // 8, 128), jnp.float32),
        mesh=mesh,
        scratch_types=[
            pltpu.VMEM((cpt, _CHUNK), jnp.int32),
            pltpu.VMEM((cpt, _CHUNK), jnp.int32),
            pltpu.VMEM((cpt, _CHUNK), jnp.float32),
            pltpu.VMEM((4, _CHUNK, 16), jnp.float32),
            pltpu.VMEM((rpt // 8, 128), jnp.float32),
            pltpu.VMEM((rpt, 16), jnp.float32),
            pltpu.VMEM_SHARED((n_pad, 16), jnp.float32),
            pltpu.VMEM_SHARED((n_pad, 16), jnp.float32),
        ] + [pltpu.SemaphoreType.DMA] * 8,
        compiler_params=pltpu.CompilerParams(use_tc_tiling_on_sc=False),
    )
    def k(ei_hbm, ew_hbm, tab_hbm, out_hbm,
          src_v, dst_v, ew_v, rows_v, b128, b16, acc_sh, tab_sh, *sems):
        gsem = sems[0:4]
        ssem = sems[4:8]
        c = lax.axis_index("c")
        s = lax.axis_index("s")
        wid = c * _NS + s
        base = wid * cpt

        def wide_to_rows(g, _):
            for l in range(16):
                b16[g * 16 + l, :] = b128[2 * g + l // 8,
                                          pl.ds(16 * (l % 8), 16)]
            return 0

        def rows_to_wide(g, _):
            for l in range(16):
                b128[2 * g + l // 8, pl.ds(16 * (l % 8), 16)] = (
                    b16[g * 16 + l, :])
            return 0

        def zrow(e, _):
            rows_v[0, e, :] = jnp.zeros((16,), jnp.float32)
            return 0
        lax.fori_loop(0, _CHUNK, zrow, 0)

        def zcopy(j, _):
            pltpu.sync_copy(rows_v.at[0],
                            acc_sh.at[pl.ds(s * rpt + j * _CHUNK, _CHUNK)])
            return 0
        lax.fori_loop(0, nzc, zcopy, 0)

        pltpu.sync_copy(tab_hbm.at[pl.ds(s * rpt // 8, rpt // 8)], b128)
        lax.fori_loop(0, rpt // 16, wide_to_rows, 0)
        pltpu.sync_copy(b16, tab_sh.at[pl.ds(s * rpt, rpt)])

        pltpu.sync_copy(ei_hbm.at[0, pl.ds(base, cpt)], src_v)
        pltpu.sync_copy(ei_hbm.at[1, pl.ds(base, cpt)], dst_v)
        pltpu.sync_copy(ew_hbm.at[pl.ds(base, cpt)], ew_v)

        plsc.subcore_barrier()

        for b in range(3):
            pltpu.async_copy(tab_sh.at[src_v.at[b]], rows_v.at[b], gsem[b])

        def quad(q, _):
            for b in range(4):
                i = 4 * q + b
                j = i + 3
                bp = (b + 3) % 4

                @pl.when(j < cpt)
                def _():
                    if b == 0:
                        @pl.when(q >= 1)
                        def _():
                            pltpu.make_async_copy(
                                rows_v.at[bp], acc_sh.at[dst_v.at[i - 1]],
                                ssem[bp]).wait()
                    else:
                        pltpu.make_async_copy(
                            rows_v.at[bp], acc_sh.at[dst_v.at[i - 1]],
                            ssem[bp]).wait()
                    pltpu.async_copy(tab_sh.at[src_v.at[j]], rows_v.at[bp],
                                     gsem[bp])

                pltpu.make_async_copy(tab_sh.at[src_v.at[i]], rows_v.at[b],
                                      gsem[b]).wait()

                def mulgrp(g, _):
                    ew16 = ew_v[i, pl.ds(g * 16, 16)]
                    for l in range(16):
                        e0 = g * 16 + l
                        rows_v[b, e0, :] = rows_v[b, e0, :] * ew16[l]
                    return 0
                lax.fori_loop(0, _CHUNK // 16, mulgrp, 0)

                pltpu.async_copy(rows_v.at[b], acc_sh.at[dst_v.at[i]],
                                 ssem[b], add=True)
            return 0
        lax.fori_loop(0, cpt // 4, quad, 0)
        for b in range(4):
            i = cpt - 4 + b
            pltpu.make_async_copy(rows_v.at[b], acc_sh.at[dst_v.at[i]],
                                  ssem[b]).wait()

        plsc.subcore_barrier()

        pltpu.sync_copy(acc_sh.at[pl.ds(s * rpt, rpt)], b16)
        lax.fori_loop(0, rpt // 16, rows_to_wide, 0)
        pltpu.sync_copy(b128,
                        out_hbm.at[pl.ds((c * n_pad + s * rpt) // 8, rpt // 8)])

    return k(ei3, ew2d, table)


def _tc1(x3, W1, rep, n_pad):
    nr = x3.shape[0]
    nrp = n_pad // 8

    def body(x_ref, w_ref, rep_ref, g_ref, sc_ref, dinv_ref):
        deg = rep_ref[0:nrp] + rep_ref[nrp:2 * nrp] + 1.0
        good = deg > 0
        dinv = jnp.where(good, lax.rsqrt(jnp.where(good, deg, 1.0)), 0.0)
        dinv_ref[...] = dinv
        if nrp > nr:
            g_ref[nr:nrp, :] = jnp.zeros((nrp - nr, 128), jnp.float32)
            sc_ref[nr:nrp, :] = jnp.zeros((nrp - nr, 128), jnp.float32)
        h = jnp.concatenate(
            [jnp.dot(x_ref[:, q, :], w_ref[...],
                     preferred_element_type=jnp.float32) for q in range(8)],
            axis=1)
        d = dinv[0:nr]
        g_ref[0:nr, :] = d * h
        sc_ref[0:nr, :] = (d * d) * h

    return pl.pallas_call(
        body,
        out_shape=[
            jax.ShapeDtypeStruct((nrp, 128), jnp.float32),
            jax.ShapeDtypeStruct((nrp, 128), jnp.float32),
            jax.ShapeDtypeStruct((nrp, 128), jnp.float32),
        ],
    )(x3, W1, rep)


def _tc2(part1, selfc1, dinv, b1exp, W2big, n_pad):
    nrp = n_pad // 8

    def body(p_ref, sc1_ref, dinv_ref, b1_ref, w2_ref, g2_ref, sc2_ref):
        dinv = dinv_ref[...]
        a1 = (dinv * (p_ref[0:nrp] + p_ref[nrp:2 * nrp])
              + sc1_ref[...] + b1_ref[...])
        z = jnp.maximum(a1, 0.0)
        h2 = jnp.dot(z, w2_ref[...], preferred_element_type=jnp.float32)
        g2_ref[...] = dinv * h2
        sc2_ref[...] = (dinv * dinv) * h2

    return pl.pallas_call(
        body,
        out_shape=[
            jax.ShapeDtypeStruct((nrp, 128), jnp.float32),
            jax.ShapeDtypeStruct((nrp, 128), jnp.float32),
        ],
    )(part1, selfc1, dinv, b1exp, W2big)


def _tc3(part2, selfc2, dinv, b2exp, n_pad, n_classes):
    nrp = n_pad // 8

    def body(q_ref, sc2_ref, dinv_ref, b2_ref, out_ref):
        t = (dinv_ref[...] * (q_ref[0:nrp] + q_ref[nrp:2 * nrp])
             + sc2_ref[...] + b2_ref[...])
        t3 = t.reshape(nrp, 8, 16)
        lane = lax.broadcasted_iota(jnp.int32, (nrp, 8, 16), 2)
        lg = jnp.where(lane < n_classes, t3, jnp.float32(-1e30))
        m = jnp.max(lg, axis=2, keepdims=True)
        e = jnp.exp(lg - m)
        lse = jnp.log(jnp.sum(e, axis=2, keepdims=True))
        out_ref[...] = (lg - m - lse).reshape(nrp, 128)

    return pl.pallas_call(
        body,
        out_shape=jax.ShapeDtypeStruct((nrp, 128), jnp.float32),
    )(part2, selfc2, dinv, b2exp)


def kernel(x, edge_index, edge_weight, W1, b1, W2, b2):
    n, d_feat = x.shape
    e = edge_index.shape[1]
    hidden = W1.shape[1]
    n_classes = W2.shape[1]

    tile_n = _NS * _CHUNK
    n_pad = -(-n // tile_n) * tile_n
    tile_e = _NW * _CHUNK * 8
    e_pad = -(-e // tile_e) * tile_e

    ei3 = jnp.pad(edge_index.astype(jnp.int32),
                  ((0, 0), (0, e_pad - e))).reshape(2, e_pad // _CHUNK, _CHUNK)
    ew = jnp.concatenate(
        [edge_weight.astype(jnp.float32), jnp.zeros((e_pad - e,), jnp.float32)])
    ew2d = ew.reshape(e_pad // _CHUNK, _CHUNK)

    b1exp = jnp.tile(b1.astype(jnp.float32), 8).reshape(1, 128)
    b2p = jnp.zeros((16,), jnp.float32).at[:n_classes].set(b2)
    b2exp = jnp.tile(b2p, 8).reshape(1, 128)
    W2p = jnp.zeros((hidden, 16), jnp.float32).at[:, :n_classes].set(W2)
    W2big = jnp.kron(jnp.eye(8, dtype=jnp.float32), W2p)

    x3 = x.astype(jnp.float32).reshape(n // 8, 8, d_feat)

    rep = _deg_sc(ei3, ew2d, n_pad)
    g1, selfc1, dinv = _tc1(x3, W1, rep, n_pad)
    part1 = _agg_sc(ei3, ew2d, g1, n_pad)
    g2, selfc2 = _tc2(part1, selfc1, dinv, b1exp, W2big, n_pad)
    part2 = _agg_sc(ei3, ew2d, g2, n_pad)
    out2d = _tc3(part2, selfc2, dinv, b2exp, n_pad, n_classes)
    out3 = out2d.reshape(n_pad // 8, 8, 16)[:n // 8, :, :n_classes]
    return out3.reshape(n, n_classes)
